# Initial kernel scaffold; baseline (speedup 1.0000x reference)
import jax
import jax.numpy as jnp
from jax.experimental import pallas as pl


def kernel(x, edge_index, raw_x, batch, W1, att_src1, att_dst1, We1, att_edge1, b1, g1, be1, W2, att_src2, att_dst2, We2, att_edge2, b2, g2, be2, Wc, bc):
    raise NotImplementedError("write your pallas kernel here")


# speedup vs baseline: 1.0907x; 1.0907x over previous
import functools
import jax
import jax.numpy as jnp
from jax.experimental import pallas as pl
from jax.experimental.pallas import tpu as pltpu

N = 10000
E = 320000
IN = 128
HEADS = 8
HD = 8
HID = 64
NC = 10
NG = 16
SIGMA = 0.5


def _gat_conv(x, src, dst, ew, W, a_s, a_d, We, a_e, bias, heads, oc):
    n = x.shape[0]
    h = (x @ W).reshape(n, heads, oc)
    al_s = (h * a_s[None, :, :]).sum(-1)
    al_d = (h * a_d[None, :, :]).sum(-1)
    ce = (We.reshape(heads, oc) * a_e).sum(-1)
    al_e = ew[:, None] * ce[None, :]
    alpha = al_s[src] + al_d[dst] + al_e
    alpha = jax.nn.leaky_relu(alpha, 0.2)
    ex = jnp.exp(alpha)
    den = jax.ops.segment_sum(ex, dst, num_segments=n)
    coef = ex / (den[dst] + 1e-16)
    msg = h[src] * coef[:, :, None]
    out = jax.ops.segment_sum(msg, dst, num_segments=n)
    return out.reshape(n, heads * oc) + bias


def _bn_elu(x, g, b):
    return jax.nn.elu((x / jnp.sqrt(1.0 + 1e-5)) * g + b)


def _tail_kernel(h_ref, batch_ref, wc_ref, bc_ref, out_ref, acc_ref, cnt_ref):
    i = pl.program_id(0)
    nb = pl.num_programs(0)

    @pl.when(i == 0)
    def _init():
        acc_ref[...] = jnp.zeros_like(acc_ref)
        cnt_ref[...] = jnp.zeros_like(cnt_ref)

    b = batch_ref[0, 0]
    h = h_ref[...]
    onehot = (b[None, :] == jax.lax.broadcasted_iota(jnp.int32, (NG, b.shape[0]), 0)).astype(jnp.float32)
    acc_ref[...] += jnp.dot(onehot, h, preferred_element_type=jnp.float32)
    cnt_ref[...] += jnp.sum(onehot, axis=1, keepdims=True)

    @pl.when(i == nb - 1)
    def _fin():
        gpool = acc_ref[...] / jnp.maximum(cnt_ref[...], 1.0)
        out_ref[...] = jnp.dot(gpool, wc_ref[...], preferred_element_type=jnp.float32) + bc_ref[...][None, :]


def _tail(h, batch, Wc, bc):
    BLK = 2000
    grid = (N // BLK,)
    return pl.pallas_call(
        _tail_kernel,
        grid=grid,
        in_specs=[
            pl.BlockSpec((BLK, HID), lambda i: (i, 0)),
            pl.BlockSpec((1, 1, BLK), lambda i: (i, 0, 0)),
            pl.BlockSpec((HID, NC), lambda i: (0, 0)),
            pl.BlockSpec((NC,), lambda i: (0,)),
        ],
        out_specs=pl.BlockSpec((NG, NC), lambda i: (0, 0)),
        out_shape=jax.ShapeDtypeStruct((NG, NC), jnp.float32),
        scratch_shapes=[pltpu.VMEM((NG, HID), jnp.float32), pltpu.VMEM((NG, 1), jnp.float32)],
    )(h, batch.reshape(N // BLK, 1, BLK), Wc, bc)


def kernel(x, edge_index, raw_x, batch, W1, att_src1, att_dst1, We1, att_edge1, b1, g1, be1, W2, att_src2, att_dst2, We2, att_edge2, b2, g2, be2, Wc, bc):
    src = edge_index[0]
    dst = edge_index[1]
    diff = raw_x[src] - raw_x[dst]
    dist2 = (diff * diff).sum(1)
    ew = jnp.exp(-dist2 / (2.0 * SIGMA * SIGMA))
    h = _gat_conv(x, src, dst, ew, W1, att_src1, att_dst1, We1, att_edge1, b1, HEADS, HD)
    h = _bn_elu(h, g1, be1)
    h = _gat_conv(h, src, dst, ew, W2, att_src2, att_dst2, We2, att_edge2, b2, 1, HID)
    h = _bn_elu(h, g2, be2)
    return _tail(h, batch, Wc, bc)


# speedup vs baseline: 19.2985x; 17.6943x over previous
import functools

import jax
import jax.numpy as jnp
from jax import lax
from jax.experimental import pallas as pl
from jax.experimental.pallas import tpu as pltpu
from jax.experimental.pallas import tpu_sc as plsc

N = 10000
E = 320000
IN = 128
HEADS = 8
HD = 8
HID = 64
NC = 10
NG = 16
SIGMA = 0.5

NP = 10240
EP = 327680
PAD_NODE = 10016
NTILES = 32
ET = EP // NTILES
STRIPE = NP // 16

CA = 2048
CHA = CA // 128
NCHA = ET // CA
CB = 512
CHB = CB // 128
NCHB = ET // CB

_MESH = dict(core_axis_name="c", subcore_axis_name="s")


def _iota16():
    return lax.broadcasted_iota(jnp.int32, (16,), 0)


def _full16(v):
    return jnp.full((16,), v, jnp.int32)


def _edge_a_call(heads, compute_ew):
    out_type = [
        jax.ShapeDtypeStruct((EP, 16), jnp.float32),
        jax.ShapeDtypeStruct((2, NP, 16), jnp.float32),
    ]
    scratch = [
        pltpu.VMEM((CHA, 128), jnp.int32),
        pltpu.VMEM((CHA, 128), jnp.int32),
        pltpu.VMEM((CA, 16), jnp.float32),
        pltpu.VMEM((CA, 16), jnp.float32),
        pltpu.VMEM((CA,), jnp.float32),
        pltpu.VMEM((heads, 16), jnp.float32),
        pltpu.SemaphoreType.DMA,
        pltpu.SemaphoreType.DMA,
    ]
    if compute_ew:
        out_type.append(jax.ShapeDtypeStruct((EP,), jnp.float32))
        scratch.append(pltpu.VMEM((3, NP), jnp.float32))
    scratch.append(pltpu.VMEM_SHARED((NP, 16), jnp.float32))

    def body(*refs):
        if compute_ew:
            (src2_r, dst2_r, al_r, ce_r, rx_r, z_r,
             ex_o, den_o, ew_o,
             idxs, idxd, asv, adv, ewv, cevv, sem, sem2,
             rxv, densh) = refs
        else:
            (src2_r, dst2_r, al_r, ce_r, ew_r, z_r,
             ex_o, den_o,
             idxs, idxd, asv, adv, ewv, cevv, sem, sem2, densh) = refs
        cid = lax.axis_index("c")
        sid = lax.axis_index("s")
        wid = cid * 16 + sid
        pltpu.sync_copy(z_r.at[pl.ds(sid * STRIPE, STRIPE)],
                        densh.at[pl.ds(sid * STRIPE, STRIPE)])
        pltpu.sync_copy(ce_r, cevv)
        if compute_ew:
            pltpu.sync_copy(rx_r, rxv)
        plsc.subcore_barrier()
        ce_b = [cevv[h] for h in range(heads)]
        i16 = _iota16()
        for ci in range(NCHA):
            row0 = wid * (ET // 128) + ci * CHA
            off = wid * ET + ci * CA
            pltpu.sync_copy(src2_r.at[pl.ds(row0, CHA)], idxs)
            pltpu.sync_copy(dst2_r.at[pl.ds(row0, CHA)], idxd)
            descs = []
            for b in range(CHA):
                descs.append(pltpu.async_copy(
                    al_r.at[idxs.at[b]], asv.at[pl.ds(b * 128, 128)], sem))
                descs.append(pltpu.async_copy(
                    al_r.at[idxd.at[b]], adv.at[pl.ds(b * 128, 128)], sem))
            if not compute_ew:
                pltpu.sync_copy(ew_r.at[pl.ds(off, CA)], ewv)
            for d in descs:
                d.wait()

            def jbody(j, carry):
                o16 = pl.multiple_of(j * 16, 16)
                rows = j * 16 + i16
                if compute_ew:
                    jr = j >> 3
                    jc = pl.multiple_of((j & 7) * 16, 16)
                    s16 = idxs[jr, pl.ds(jc, 16)]
                    d16 = idxd[jr, pl.ds(jc, 16)]
                    dacc = None
                    for k in range(3):
                        kf = _full16(k)
                        df = (plsc.load_gather(rxv, [kf, s16])
                              - plsc.load_gather(rxv, [kf, d16]))
                        dacc = df * df if dacc is None else dacc + df * df
                    ew16 = jnp.exp(dacc * (-1.0 / (2.0 * SIGMA * SIGMA)))
                    ewv[pl.ds(o16, 16)] = ew16
                else:
                    ew16 = ewv[pl.ds(o16, 16)]
                for h in range(heads):
                    a = (plsc.load_gather(asv, [rows, _full16(h)])
                         + plsc.load_gather(adv, [rows, _full16(h + 8)])
                         + ce_b[h] * ew16)
                    a = jnp.maximum(a, a * 0.2)
                    plsc.store_scatter(asv, [rows, _full16(h)], jnp.exp(a))
                return carry

            lax.fori_loop(0, CA // 16, jbody, 0)
            descs = []
            for b in range(CHA):
                descs.append(pltpu.async_copy(
                    asv.at[pl.ds(b * 128, 128)], densh.at[idxd.at[b]], sem2,
                    add=True))
            for d in descs:
                d.wait()
            pltpu.sync_copy(asv, ex_o.at[pl.ds(off, CA)])
            if compute_ew:
                pltpu.sync_copy(ewv, ew_o.at[pl.ds(off, CA)])
        plsc.subcore_barrier()
        pltpu.sync_copy(densh.at[pl.ds(sid * STRIPE, STRIPE)],
                        den_o.at[cid, pl.ds(sid * STRIPE, STRIPE)])

    return pl.kernel(
        body,
        out_type=tuple(out_type),
        mesh=plsc.VectorSubcoreMesh(**_MESH),
        scratch_types=tuple(scratch),
        compiler_params=pltpu.CompilerParams(needs_layout_passes=False, use_tc_tiling_on_sc=False),
    )


def _edge_b_call(heads):
    fan = 64 // heads
    out_type = jax.ShapeDtypeStruct((2, NP, 64), jnp.float32)
    scratch = (
        pltpu.VMEM((CHB, 128), jnp.int32),
        pltpu.VMEM((CHB, 128), jnp.int32),
        pltpu.VMEM((CB, 64), jnp.float32),
        pltpu.VMEM((CB, 16), jnp.float32),
        pltpu.VMEM((CB, 16), jnp.float32),
        pltpu.SemaphoreType.DMA,
        pltpu.SemaphoreType.DMA,
        pltpu.VMEM_SHARED((NP, 64), jnp.float32),
    )

    def body(src2_r, dst2_r, h_r, ex_r, rd_r, z_r, acc_o,
             idxs, idxd, hv, rdv, exv, sem, sem2, accsh):
        cid = lax.axis_index("c")
        sid = lax.axis_index("s")
        wid = cid * 16 + sid
        pltpu.sync_copy(z_r.at[pl.ds(sid * STRIPE, STRIPE)],
                        accsh.at[pl.ds(sid * STRIPE, STRIPE)])
        plsc.subcore_barrier()
        i16 = _iota16()
        for ci in range(NCHB):
            row0 = wid * (ET // 128) + ci * CHB
            off = wid * ET + ci * CB
            pltpu.sync_copy(src2_r.at[pl.ds(row0, CHB)], idxs)
            pltpu.sync_copy(dst2_r.at[pl.ds(row0, CHB)], idxd)
            descs = []
            for b in range(CHB):
                descs.append(pltpu.async_copy(
                    h_r.at[idxs.at[b]], hv.at[pl.ds(b * 128, 128)], sem))
                descs.append(pltpu.async_copy(
                    rd_r.at[idxd.at[b]], rdv.at[pl.ds(b * 128, 128)], sem))
            pltpu.sync_copy(ex_r.at[pl.ds(off, CB)], exv)
            for d in descs:
                d.wait()

            def jbody(j, carry):
                rows = j * 16 + i16
                for h in range(heads):
                    hf = _full16(h)
                    cr = (plsc.load_gather(exv, [rows, hf])
                          * plsc.load_gather(rdv, [rows, hf]))
                    for d in range(fan):
                        cf = _full16(h * fan + d)
                        v = plsc.load_gather(hv, [rows, cf]) * cr
                        plsc.store_scatter(hv, [rows, cf], v)
                return carry

            lax.fori_loop(0, CB // 16, jbody, 0)
            descs = []
            for b in range(CHB):
                descs.append(pltpu.async_copy(
                    hv.at[pl.ds(b * 128, 128)], accsh.at[idxd.at[b]], sem2,
                    add=True))
            for d in descs:
                d.wait()
        plsc.subcore_barrier()
        pltpu.sync_copy(accsh.at[pl.ds(sid * STRIPE, STRIPE)],
                        acc_o.at[cid, pl.ds(sid * STRIPE, STRIPE)])

    return pl.kernel(
        body,
        out_type=out_type,
        mesh=plsc.VectorSubcoreMesh(**_MESH),
        scratch_types=scratch,
        compiler_params=pltpu.CompilerParams(needs_layout_passes=False, use_tc_tiling_on_sc=False),
    )


def _prep_body(x_ref, w_ref, a_ref, h_ref, al_ref):
    h = jnp.dot(x_ref[...], w_ref[...], preferred_element_type=jnp.float32)
    h_ref[...] = h
    al_ref[...] = jnp.dot(h, a_ref[...], preferred_element_type=jnp.float32)


def _tc_prep(xp, W, A):
    BLK = 2048
    return pl.pallas_call(
        _prep_body,
        grid=(NP // BLK,),
        in_specs=[
            pl.BlockSpec((BLK, xp.shape[1]), lambda i: (i, 0)),
            pl.BlockSpec(W.shape, lambda i: (0, 0)),
            pl.BlockSpec(A.shape, lambda i: (0, 0)),
        ],
        out_specs=[
            pl.BlockSpec((BLK, W.shape[1]), lambda i: (i, 0)),
            pl.BlockSpec((BLK, 16), lambda i: (i, 0)),
        ],
        out_shape=[
            jax.ShapeDtypeStruct((NP, W.shape[1]), jnp.float32),
            jax.ShapeDtypeStruct((NP, 16), jnp.float32),
        ],
    )(xp, W, A)


def _recip_body(d_ref, r_ref):
    r_ref[...] = 1.0 / (d_ref[0] + d_ref[1] + 1e-16)


def _tc_recip(den):
    return pl.pallas_call(
        _recip_body,
        out_shape=jax.ShapeDtypeStruct((NP, 16), jnp.float32),
    )(den)


_BN_SCALE = float((1.0 + 1e-5) ** -0.5)


def _mid_body(acc_ref, b_ref, g_ref, be_ref, w_ref, a_ref, h_ref, al_ref):
    s = acc_ref[0] + acc_ref[1] + b_ref[...][None, :]
    s = s * (g_ref[...] * _BN_SCALE)[None, :] + be_ref[...][None, :]
    s = jnp.where(s > 0, s, jnp.exp(s) - 1.0)
    h = jnp.dot(s, w_ref[...], preferred_element_type=jnp.float32)
    h_ref[...] = h
    al_ref[...] = jnp.dot(h, a_ref[...], preferred_element_type=jnp.float32)


def _tc_mid(acc, b, g, be, W, A):
    BLK = 2048
    return pl.pallas_call(
        _mid_body,
        grid=(NP // BLK,),
        in_specs=[
            pl.BlockSpec((2, BLK, 64), lambda i: (0, i, 0)),
            pl.BlockSpec((64,), lambda i: (0,)),
            pl.BlockSpec((64,), lambda i: (0,)),
            pl.BlockSpec((64,), lambda i: (0,)),
            pl.BlockSpec((64, 64), lambda i: (0, 0)),
            pl.BlockSpec((64, 16), lambda i: (0, 0)),
        ],
        out_specs=[
            pl.BlockSpec((BLK, 64), lambda i: (i, 0)),
            pl.BlockSpec((BLK, 16), lambda i: (i, 0)),
        ],
        out_shape=[
            jax.ShapeDtypeStruct((NP, 64), jnp.float32),
            jax.ShapeDtypeStruct((NP, 16), jnp.float32),
        ],
    )(acc, b, g, be, W, A)


def _tail_body(acc_ref, b_ref, g_ref, be_ref, batch_ref, wc_ref, bc_ref,
               out_ref, pool_ref, cnt_ref):
    i = pl.program_id(0)
    nb = pl.num_programs(0)

    @pl.when(i == 0)
    def _init():
        pool_ref[...] = jnp.zeros_like(pool_ref)
        cnt_ref[...] = jnp.zeros_like(cnt_ref)

    s = acc_ref[0] + acc_ref[1] + b_ref[...][None, :]
    s = s * (g_ref[...] * _BN_SCALE)[None, :] + be_ref[...][None, :]
    h = jnp.where(s > 0, s, jnp.exp(s) - 1.0)
    bvec = batch_ref[0, 0]
    onehot = (bvec[None, :] == lax.broadcasted_iota(
        jnp.int32, (NG, bvec.shape[0]), 0)).astype(jnp.float32)
    pool_ref[...] += jnp.dot(onehot, h, preferred_element_type=jnp.float32)
    cnt_ref[...] += jnp.sum(onehot, axis=1, keepdims=True)

    @pl.when(i == nb - 1)
    def _fin():
        gpool = pool_ref[...] / jnp.maximum(cnt_ref[...], 1.0)
        out_ref[...] = jnp.dot(gpool, wc_ref[...],
                               preferred_element_type=jnp.float32) + bc_ref[...][None, :]


def _tc_tail(acc, b, g, be, batch, Wc, bc):
    BLK = 2000
    return pl.pallas_call(
        _tail_body,
        grid=(N // BLK,),
        in_specs=[
            pl.BlockSpec((2, BLK, 64), lambda i: (0, i, 0)),
            pl.BlockSpec((64,), lambda i: (0,)),
            pl.BlockSpec((64,), lambda i: (0,)),
            pl.BlockSpec((64,), lambda i: (0,)),
            pl.BlockSpec((1, 1, BLK), lambda i: (i, 0, 0)),
            pl.BlockSpec((HID, NC), lambda i: (0, 0)),
            pl.BlockSpec((NC,), lambda i: (0,)),
        ],
        out_specs=pl.BlockSpec((NG, NC), lambda i: (0, 0)),
        out_shape=jax.ShapeDtypeStruct((NG, NC), jnp.float32),
        scratch_shapes=[pltpu.VMEM((NG, HID), jnp.float32),
                        pltpu.VMEM((NG, 1), jnp.float32)],
    )(acc, b, g, be, batch.reshape(N // BLK, 1, BLK), Wc, bc)


def kernel(x, edge_index, raw_x, batch, W1, att_src1, att_dst1, We1,
           att_edge1, b1, g1, be1, W2, att_src2, att_dst2, We2, att_edge2,
           b2, g2, be2, Wc, bc):
    src = edge_index[0]
    dst = edge_index[1]
    pad = jnp.full((EP - E,), PAD_NODE, jnp.int32)
    srcp = jnp.concatenate([src, pad])
    dstp = jnp.concatenate([dst, pad])
    src2d = srcp.reshape(EP // 128, 128)
    dst2d = dstp.reshape(EP // 128, 128)

    xp = jnp.pad(x, ((0, NP - N), (0, 0)))
    rx = jnp.pad(raw_x, ((0, NP - N), (0, 0))).T
    z16 = jnp.zeros((NP, 16), jnp.float32)
    z64 = jnp.zeros((NP, 64), jnp.float32)

    eye8 = jnp.eye(8, dtype=jnp.float32)
    As1 = (eye8[:, None, :] * att_src1[:, :, None]).reshape(64, 8)
    Ad1 = (eye8[:, None, :] * att_dst1[:, :, None]).reshape(64, 8)
    A1 = jnp.concatenate([As1, Ad1], axis=1)
    ce1 = (We1.reshape(HEADS, HD) * att_edge1).sum(-1)
    cev1 = jnp.repeat(ce1[:, None], 16, axis=1)

    A2 = jnp.zeros((64, 16), jnp.float32)
    A2 = A2.at[:, 0].set(att_src2[0]).at[:, 8].set(att_dst2[0])
    ce2 = (We2[0] * att_edge2[0]).sum()
    cev2 = jnp.full((1, 16), ce2, jnp.float32)

    h1, AL1 = _tc_prep(xp, W1, A1)
    EX1, den1, ew = _edge_a_call(HEADS, True)(
        src2d, dst2d, AL1, cev1, rx, z16)
    rden1 = _tc_recip(den1)
    acc1 = _edge_b_call(HEADS)(src2d, dst2d, h1, EX1, rden1, z64)
    h2, AL2 = _tc_mid(acc1, b1, g1, be1, W2, A2)
    EX2, den2 = _edge_a_call(1, False)(src2d, dst2d, AL2, cev2, ew, z16)
    rden2 = _tc_recip(den2)
    acc2 = _edge_b_call(1)(src2d, dst2d, h2, EX2, rden2, z64)
    return _tc_tail(acc2, b2, g2, be2, batch, Wc, bc)
